# baseline (speedup 1.0000x reference)
import functools

import jax
import jax.numpy as jnp
from jax import lax
from jax.experimental import pallas as pl
from jax.experimental.pallas import tpu as pltpu
from jax.experimental.pallas import tpu_sc as plsc

DIM = 64
B = 16384
NC = 2
NS = 16
NW = NC * NS
BPW = B // NW
CHUNK = 128
NCHUNK = BPW // CHUNK
LANES = 16


def _make_sc_dot():
    mesh = plsc.VectorSubcoreMesh(core_axis_name="c", subcore_axis_name="s")

    @functools.partial(
        pl.kernel,
        mesh=mesh,
        out_type=jax.ShapeDtypeStruct((B,), jnp.float32),
        scratch_types=[
            pltpu.VMEM((NCHUNK, CHUNK), jnp.int32),
            pltpu.VMEM((NCHUNK, CHUNK), jnp.int32),
            pltpu.VMEM((BPW, DIM), jnp.float32),
            pltpu.VMEM((BPW, DIM), jnp.float32),
            pltpu.VMEM((BPW,), jnp.float32),
            pltpu.VMEM((LANES, LANES + 1), jnp.float32),
            pltpu.SemaphoreType.DMA,
            pltpu.SemaphoreType.DMA,
        ],
        compiler_params=pltpu.CompilerParams(
            needs_layout_passes=False, use_tc_tiling_on_sc=False),
    )
    def sc_dot(idx1_hbm, idx2_hbm, emb_hbm, ctx_hbm, out_hbm,
               idx1_v, idx2_v, rows1_v, rows2_v, dot_v, pbuf_v, sem1, sem2):
        wid = lax.axis_index("s") * NC + lax.axis_index("c")
        pltpu.sync_copy(idx1_hbm.at[wid], idx1_v)
        pltpu.sync_copy(idx2_hbm.at[wid], idx2_v)
        copies = []
        for j in range(NCHUNK):
            copies.append(pltpu.async_copy(
                emb_hbm.at[idx1_v.at[j]],
                rows1_v.at[pl.ds(j * CHUNK, CHUNK)], sem1))
            copies.append(pltpu.async_copy(
                ctx_hbm.at[idx2_v.at[j]],
                rows2_v.at[pl.ds(j * CHUNK, CHUNK)], sem2))
        for c in copies:
            c.wait()

        lane = jnp.arange(LANES, dtype=jnp.int32)

        def group(g, carry):
            base = g * LANES
            for j in range(LANES):
                i = base + j
                acc = rows1_v[i, pl.ds(0, LANES)] * rows2_v[i, pl.ds(0, LANES)]
                for k in range(1, DIM // LANES):
                    acc = acc + (rows1_v[i, pl.ds(k * LANES, LANES)]
                                 * rows2_v[i, pl.ds(k * LANES, LANES)])
                pbuf_v[j, pl.ds(0, LANES)] = acc
            dot = jnp.zeros((LANES,), jnp.float32)
            for k in range(LANES):
                col = plsc.load_gather(
                    pbuf_v, [lane, jnp.full((LANES,), k, jnp.int32)])
                dot = dot + col
            dot_v[pl.ds(base, LANES)] = dot
            return carry

        lax.fori_loop(0, BPW // LANES, group, 0)
        pltpu.sync_copy(dot_v, out_hbm.at[pl.ds(wid * BPW, BPW)])

    return sc_dot


_SC_DOT = _make_sc_dot()


def _tc_loss_body(dot_ref, tgt_ref, out_ref):
    x = jax.nn.sigmoid(dot_ref[...])
    t = tgt_ref[...]
    l = jnp.clip(x, 0.0, None) - x * t + jnp.log1p(jnp.exp(-jnp.abs(x)))
    out_ref[...] = (jnp.sum(l) * (1.0 / B)).reshape(1, 1)


_TC_LOSS = pl.pallas_call(
    _tc_loss_body,
    out_shape=jax.ShapeDtypeStruct((1, 1), jnp.float32),
)


def kernel(word1_index, word2_index, target, emb_table, ctx_table):
    idx1 = word1_index.astype(jnp.int32).reshape(NW, NCHUNK, CHUNK)
    idx2 = word2_index.astype(jnp.int32).reshape(NW, NCHUNK, CHUNK)
    dot = _SC_DOT(idx1, idx2, emb_table, ctx_table)
    loss = _TC_LOSS(dot.reshape(128, 128), target.reshape(128, 128))
    return loss[0, 0]


# speedup vs baseline: 2.2381x; 2.2381x over previous
import functools

import jax
import jax.numpy as jnp
from jax import lax
from jax.experimental import pallas as pl
from jax.experimental.pallas import tpu as pltpu
from jax.experimental.pallas import tpu_sc as plsc

VOCAB = 1000000
DIM = 64
B = 16384
NC = 2
NS = 16
NW = NC * NS
BPW = B // NW
CH = 128
NCHUNK = BPW // CH
KFIRE = 32
LANES = 16


def _make_sc_dot():
    mesh = plsc.VectorSubcoreMesh(core_axis_name="c", subcore_axis_name="s")

    @functools.partial(
        pl.kernel,
        mesh=mesh,
        out_type=jax.ShapeDtypeStruct((B,), jnp.float32),
        scratch_types=[
            pltpu.VMEM((BPW,), jnp.int32),
            pltpu.VMEM((BPW,), jnp.int32),
            pltpu.VMEM((CH, DIM), jnp.float32),
            pltpu.VMEM((CH, DIM), jnp.float32),
            pltpu.VMEM((BPW,), jnp.float32),
            pltpu.VMEM((LANES, LANES + 1), jnp.float32),
            pltpu.SemaphoreType.DMA,
            pltpu.SemaphoreType.DMA,
        ],
        compiler_params=pltpu.CompilerParams(needs_layout_passes=False),
    )
    def sc_dot(idx1_hbm, idx2_hbm, emb_hbm, ctx_hbm, out_hbm,
               idx1_v, idx2_v, rows1_v, rows2_v, dot_v, pbuf_v, sem1, sem2):
        wid = lax.axis_index("s") * NC + lax.axis_index("c")
        base = wid * BPW
        pltpu.sync_copy(idx1_hbm.at[pl.ds(base, BPW)], idx1_v)
        pltpu.sync_copy(idx2_hbm.at[pl.ds(base, BPW)], idx2_v)
        lane = jnp.arange(LANES, dtype=jnp.int32)

        def chunk(c, carry):
            cbase = c * CH
            for b in range(CH // LANES):
                iv1 = idx1_v[pl.ds(cbase + b * LANES, LANES)]
                iv2 = idx2_v[pl.ds(cbase + b * LANES, LANES)]
                s1, r1 = iv1 >> 3, iv1 & 7
                s2, r2 = iv2 >> 3, iv2 & 7
                copies = []
                for jj in range(LANES):
                    j = b * LANES + jj
                    copies.append(pltpu.async_copy(
                        emb_hbm.at[s1[jj], r1[jj]], rows1_v.at[j], sem1))
                    copies.append(pltpu.async_copy(
                        ctx_hbm.at[s2[jj], r2[jj]], rows2_v.at[j], sem2))
                for cp in copies:
                    cp.wait()
            for g in range(CH // LANES):
                for j in range(LANES):
                    i = g * LANES + j
                    acc = rows1_v[i, pl.ds(0, LANES)] * rows2_v[i, pl.ds(0, LANES)]
                    for k in range(1, DIM // LANES):
                        acc = acc + (rows1_v[i, pl.ds(k * LANES, LANES)]
                                     * rows2_v[i, pl.ds(k * LANES, LANES)])
                    pbuf_v[j, pl.ds(0, LANES)] = acc
                dot = jnp.zeros((LANES,), jnp.float32)
                for k in range(LANES):
                    col = plsc.load_gather(
                        pbuf_v, [lane, jnp.full((LANES,), k, jnp.int32)])
                    dot = dot + col
                dot_v[pl.ds(cbase + g * LANES, LANES)] = dot
            return carry

        lax.fori_loop(0, NCHUNK, chunk, 0)
        pltpu.sync_copy(dot_v, out_hbm.at[pl.ds(base, BPW)])

    return sc_dot


_SC_DOT = _make_sc_dot()


def _tc_loss_body(dot_ref, tgt_ref, out_ref):
    x = jax.nn.sigmoid(dot_ref[...])
    t = tgt_ref[...]
    l = jnp.clip(x, 0.0, None) - x * t + jnp.log1p(jnp.exp(-jnp.abs(x)))
    out_ref[...] = (jnp.sum(l) * (1.0 / B)).reshape(1, 1)


_TC_LOSS = pl.pallas_call(
    _tc_loss_body,
    out_shape=jax.ShapeDtypeStruct((1, 1), jnp.float32),
)


def kernel(word1_index, word2_index, target, emb_table, ctx_table):
    idx1 = word1_index.astype(jnp.int32)
    idx2 = word2_index.astype(jnp.int32)
    emb3 = emb_table.reshape(VOCAB // 8, 8, DIM)
    ctx3 = ctx_table.reshape(VOCAB // 8, 8, DIM)
    dot = _SC_DOT(idx1, idx2, emb3, ctx3)
    loss = _TC_LOSS(dot.reshape(128, 128), target.reshape(128, 128))
    return loss[0, 0]
